# Initial kernel scaffold; baseline (speedup 1.0000x reference)
import jax
import jax.numpy as jnp
from jax.experimental import pallas as pl


def kernel(node_feats, edge_index, enc_W, enc_b, att_W0, att_b0, t_W0, t_b0, att_W1, att_b1, t_W1, t_b1, r1_W, r1_b, r2_W, r2_b, e1_W, e1_b, e2_W, e2_b):
    raise NotImplementedError("write your pallas kernel here")


# speedup vs baseline: 2.6504x; 2.6504x over previous
import functools

import jax
import jax.numpy as jnp
from jax import lax
from jax.experimental import pallas as pl
from jax.experimental.pallas import tpu as pltpu
from jax.experimental.pallas import tpu_sc as plsc

N = 10000
D = 128
E = 320000

NC = 2
NS = 16
L = 16
NW = NC * NS
CH = 128
EPT = 10240
E_PAD = NW * EPT
NCHUNK = EPT // CH
N_PAD = 10240
RPT = N_PAD // NS

_MESH = plsc.VectorSubcoreMesh(core_axis_name="c", subcore_axis_name="s")
_SC_PARAMS = pltpu.CompilerParams(needs_layout_passes=False)

ROWS_BLK = 1000
GRID_N = N // ROWS_BLK


def _wid():
    return lax.axis_index("s") * NC + lax.axis_index("c")


def _encode_body(x_ref, w_ref, b_ref, w2_ref, b2_ref, h_ref, a_ref):
    h = jnp.maximum(
        jnp.dot(x_ref[...], w_ref[...], preferred_element_type=jnp.float32)
        + b_ref[...], 0.0)
    h_ref[...] = h
    a_ref[...] = (jnp.dot(h, w2_ref[...], preferred_element_type=jnp.float32)
                  + b2_ref[...])


def _encode(x, w, b, w2, b2):
    return pl.pallas_call(
        _encode_body,
        grid=(GRID_N,),
        in_specs=[
            pl.BlockSpec((ROWS_BLK, D), lambda i: (i, 0)),
            pl.BlockSpec((D, D), lambda i: (0, 0)),
            pl.BlockSpec((1, D), lambda i: (0, 0)),
            pl.BlockSpec((D, D), lambda i: (0, 0)),
            pl.BlockSpec((1, D), lambda i: (0, 0)),
        ],
        out_specs=[
            pl.BlockSpec((ROWS_BLK, D), lambda i: (i, 0)),
            pl.BlockSpec((ROWS_BLK, D), lambda i: (i, 0)),
        ],
        out_shape=[
            jax.ShapeDtypeStruct((N, D), jnp.float32),
            jax.ShapeDtypeStruct((N, D), jnp.float32),
        ],
    )(x, w, b, w2, b2)


def _layer_body(p0_ref, p1_ref, w_ref, b_ref, hp_ref, w2_ref, b2_ref,
                h_ref, a_ref):
    agg = p0_ref[...] + p1_ref[...]
    h = jnp.maximum(
        jnp.dot(agg, w_ref[...], preferred_element_type=jnp.float32)
        + b_ref[...], 0.0) + hp_ref[...]
    h_ref[...] = h
    a_ref[...] = (jnp.dot(h, w2_ref[...], preferred_element_type=jnp.float32)
                  + b2_ref[...])


def _layer(p0, p1, w, b, hp, w2, b2):
    return pl.pallas_call(
        _layer_body,
        grid=(GRID_N,),
        in_specs=[
            pl.BlockSpec((ROWS_BLK, D), lambda i: (i, 0)),
            pl.BlockSpec((ROWS_BLK, D), lambda i: (i, 0)),
            pl.BlockSpec((D, D), lambda i: (0, 0)),
            pl.BlockSpec((1, D), lambda i: (0, 0)),
            pl.BlockSpec((ROWS_BLK, D), lambda i: (i, 0)),
            pl.BlockSpec((D, D), lambda i: (0, 0)),
            pl.BlockSpec((1, D), lambda i: (0, 0)),
        ],
        out_specs=[
            pl.BlockSpec((ROWS_BLK, D), lambda i: (i, 0)),
            pl.BlockSpec((ROWS_BLK, D), lambda i: (i, 0)),
        ],
        out_shape=[
            jax.ShapeDtypeStruct((N, D), jnp.float32),
            jax.ShapeDtypeStruct((N, D), jnp.float32),
        ],
    )(p0, p1, w, b, hp, w2, b2)


def _final_body(p0_ref, p1_ref, w_ref, b_ref, hp_ref, wu_ref, bu_ref, wv_ref,
                h_ref, u_ref, v_ref, pool_ref):
    agg = p0_ref[...] + p1_ref[...]
    h = jnp.maximum(
        jnp.dot(agg, w_ref[...], preferred_element_type=jnp.float32)
        + b_ref[...], 0.0) + hp_ref[...]
    h_ref[...] = h
    u_ref[...] = (jnp.dot(h, wu_ref[...], preferred_element_type=jnp.float32)
                  + bu_ref[...])
    v_ref[...] = jnp.dot(h, wv_ref[...], preferred_element_type=jnp.float32)

    @pl.when(pl.program_id(0) == 0)
    def _():
        pool_ref[...] = jnp.zeros_like(pool_ref)

    pool_ref[...] += jnp.sum(h, axis=0, keepdims=True)


def _final(p0, p1, w, b, hp, wu, bu, wv):
    return pl.pallas_call(
        _final_body,
        grid=(GRID_N,),
        in_specs=[
            pl.BlockSpec((ROWS_BLK, D), lambda i: (i, 0)),
            pl.BlockSpec((ROWS_BLK, D), lambda i: (i, 0)),
            pl.BlockSpec((D, D), lambda i: (0, 0)),
            pl.BlockSpec((1, D), lambda i: (0, 0)),
            pl.BlockSpec((ROWS_BLK, D), lambda i: (i, 0)),
            pl.BlockSpec((D, D), lambda i: (0, 0)),
            pl.BlockSpec((1, D), lambda i: (0, 0)),
            pl.BlockSpec((D, D), lambda i: (0, 0)),
        ],
        out_specs=[
            pl.BlockSpec((ROWS_BLK, D), lambda i: (i, 0)),
            pl.BlockSpec((ROWS_BLK, D), lambda i: (i, 0)),
            pl.BlockSpec((ROWS_BLK, D), lambda i: (i, 0)),
            pl.BlockSpec((1, D), lambda i: (0, 0)),
        ],
        out_shape=[
            jax.ShapeDtypeStruct((N, D), jnp.float32),
            jax.ShapeDtypeStruct((N, D), jnp.float32),
            jax.ShapeDtypeStruct((N, D), jnp.float32),
            jax.ShapeDtypeStruct((1, D), jnp.float32),
        ],
    )(p0, p1, w, b, hp, wu, bu, wv)


def _softmax_body(w_ref, att_ref):
    w = w_ref[...]
    m = jnp.max(w)
    e = jnp.exp(w - m)
    att_ref[...] = e / jnp.sum(e)


def _softmax(w2d):
    return pl.pallas_call(
        _softmax_body,
        out_shape=jax.ShapeDtypeStruct(w2d.shape, jnp.float32),
    )(w2d)


def _softmax2_body(w_ref, att0_ref, att_ref, asum_ref):
    w = w_ref[...]
    m = jnp.max(w)
    e = jnp.exp(w - m)
    att = e / jnp.sum(e)
    att_ref[...] = att
    asum_ref[...] = (att + att0_ref[...]) * 0.5


def _softmax2(w2d, att0):
    return pl.pallas_call(
        _softmax2_body,
        out_shape=[
            jax.ShapeDtypeStruct(w2d.shape, jnp.float32),
            jax.ShapeDtypeStruct(w2d.shape, jnp.float32),
        ],
    )(w2d, att0)


def _readout_body(pool_ref, r1w_ref, r1b_ref, r2w_ref, r2b_ref, out_ref):
    pooled = pool_ref[...] * (1.0 / N)
    q = jnp.maximum(
        jnp.dot(pooled, r1w_ref[...], preferred_element_type=jnp.float32)
        + r1b_ref[...], 0.0)
    s = jnp.sum(q * r2w_ref[...]) + r2b_ref[0, 0]
    out_ref[...] = jnp.full((1, 1), s, jnp.float32)


def _readout(pool, r1w, r1b, r2w_row, r2b):
    return pl.pallas_call(
        _readout_body,
        out_shape=jax.ShapeDtypeStruct((1, 1), jnp.float32),
    )(pool, r1w, r1b, r2w_row, r2b)


@functools.partial(
    pl.kernel,
    out_type=jax.ShapeDtypeStruct((E_PAD,), jnp.float32),
    mesh=_MESH,
    compiler_params=_SC_PARAMS,
    scratch_types=[
        pltpu.VMEM((N,), jnp.float32),
        pltpu.VMEM((N,), jnp.float32),
        pltpu.VMEM((CH,), jnp.int32),
        pltpu.VMEM((CH,), jnp.int32),
        pltpu.VMEM((CH,), jnp.float32),
    ],
)
def _edge_scores(asrc_hbm, adst_hbm, src_hbm, dst_hbm, w_hbm,
                 as_v, ad_v, si_v, di_v, w_v):
    wid = _wid()
    pltpu.sync_copy(asrc_hbm, as_v)
    pltpu.sync_copy(adst_hbm, ad_v)

    def chunk(i, carry):
        base = wid * EPT + i * CH
        pltpu.sync_copy(src_hbm.at[pl.ds(base, CH)], si_v)
        pltpu.sync_copy(dst_hbm.at[pl.ds(base, CH)], di_v)
        for g in range(CH // L):
            sl = pl.ds(g * L, L)
            x = (plsc.load_gather(as_v, [si_v[sl]])
                 + plsc.load_gather(ad_v, [di_v[sl]]))
            w_v[sl] = jnp.where(x >= 0.0, x, 0.2 * x)
        pltpu.sync_copy(w_v, w_hbm.at[pl.ds(base, CH)])
        return carry

    lax.fori_loop(0, NCHUNK, chunk, 0)


@functools.partial(
    pl.kernel,
    out_type=jax.ShapeDtypeStruct((NC, N_PAD, D), jnp.float32),
    mesh=_MESH,
    compiler_params=_SC_PARAMS,
    scratch_types=[
        pltpu.VMEM((CH,), jnp.int32),
        pltpu.VMEM((CH,), jnp.int32),
        pltpu.VMEM((CH,), jnp.float32),
        pltpu.VMEM((CH, D), jnp.float32),
        pltpu.VMEM_SHARED((N_PAD, D), jnp.float32),
        pltpu.SemaphoreType.DMA,
    ],
)
def _spmm(h_hbm, src_hbm, dst_hbm, att_hbm, zer_hbm, out_hbm,
          si_v, di_v, at_v, rows_v, acc_sh, sem):
    cid = lax.axis_index("c")
    sid = lax.axis_index("s")
    wid = sid * NC + cid

    pltpu.sync_copy(zer_hbm.at[pl.ds(sid * RPT, RPT)],
                    acc_sh.at[pl.ds(sid * RPT, RPT)])
    plsc.subcore_barrier()

    def chunk(i, carry):
        base = wid * EPT + i * CH
        pltpu.sync_copy(src_hbm.at[pl.ds(base, CH)], si_v)
        pltpu.sync_copy(dst_hbm.at[pl.ds(base, CH)], di_v)
        pltpu.sync_copy(att_hbm.at[pl.ds(base, CH)], at_v)
        pltpu.async_copy(h_hbm.at[si_v], rows_v, sem).wait()

        def scale(g, c2):
            av = at_v[pl.ds(g * L, L)]
            for t in range(L):
                e = g * L + t
                a = av[t]
                for j in range(D // L):
                    sl = pl.ds(j * L, L)
                    rows_v[e, sl] = rows_v[e, sl] * a
            return c2

        lax.fori_loop(0, CH // L, scale, 0)
        pltpu.sync_copy(rows_v, acc_sh.at[di_v], add=True)
        return carry

    lax.fori_loop(0, NCHUNK, chunk, 0)
    plsc.subcore_barrier()
    pltpu.sync_copy(acc_sh.at[pl.ds(sid * RPT, RPT)],
                    out_hbm.at[cid, pl.ds(sid * RPT, RPT)])


@functools.partial(
    pl.kernel,
    out_type=jax.ShapeDtypeStruct((E_PAD,), jnp.float32),
    mesh=_MESH,
    compiler_params=_SC_PARAMS,
    scratch_types=[
        pltpu.VMEM((CH,), jnp.int32),
        pltpu.VMEM((CH,), jnp.int32),
        pltpu.VMEM((CH,), jnp.float32),
        pltpu.VMEM((CH, D), jnp.float32),
        pltpu.VMEM((CH, D), jnp.float32),
        pltpu.VMEM((D,), jnp.float32),
        pltpu.VMEM((L,), jnp.float32),
        pltpu.VMEM((CH,), jnp.float32),
        pltpu.SemaphoreType.DMA,
        pltpu.SemaphoreType.DMA,
    ],
)
def _edge_imp(u_hbm, v_hbm, src_hbm, dst_hbm, asum_hbm, w2_hbm, e2b_hbm,
              out_hbm, si_v, di_v, as_v, ur_v, vr_v, w2_v, e2b_v, o_v,
              sem1, sem2):
    wid = _wid()
    pltpu.sync_copy(w2_hbm, w2_v)
    pltpu.sync_copy(e2b_hbm, e2b_v)
    iotas = [jnp.arange(L, dtype=jnp.int32) == t for t in range(L)]

    def chunk(i, carry):
        base = wid * EPT + i * CH
        pltpu.sync_copy(src_hbm.at[pl.ds(base, CH)], si_v)
        pltpu.sync_copy(dst_hbm.at[pl.ds(base, CH)], di_v)
        pltpu.sync_copy(asum_hbm.at[pl.ds(base, CH)], as_v)
        cp1 = pltpu.async_copy(u_hbm.at[si_v], ur_v, sem1)
        cp2 = pltpu.async_copy(v_hbm.at[di_v], vr_v, sem2)
        cp1.wait()
        cp2.wait()

        def group(g, c2):
            sums = jnp.zeros((L,), jnp.float32)
            for t in range(L):
                e = g * L + t
                acc = jnp.zeros((L,), jnp.float32)
                for j in range(D // L):
                    sl = pl.ds(j * L, L)
                    z = jnp.maximum(ur_v[e, sl] + vr_v[e, sl], 0.0)
                    acc = acc + z * w2_v[sl]
                sums = jnp.where(iotas[t], jnp.sum(acc), sums)
            x = sums + e2b_v[...]
            p = 1.0 / (1.0 + jnp.exp(-x))
            gsl = pl.ds(g * L, L)
            o_v[gsl] = p * as_v[gsl]
            return c2

        lax.fori_loop(0, CH // L, group, 0)
        pltpu.sync_copy(o_v, out_hbm.at[pl.ds(base, CH)])
        return carry

    lax.fori_loop(0, NCHUNK, chunk, 0)


def _att_proj_mats(att_W, att_b):
    w2 = jnp.zeros((D, D), jnp.float32)
    w2 = w2.at[:, 0].set(att_W[:D, 0]).at[:, 1].set(att_W[D:, 0])
    b2 = jnp.zeros((1, D), jnp.float32).at[0, 0].set(att_b[0])
    return w2, b2


def kernel(node_feats, edge_index, enc_W, enc_b,
           att_W0, att_b0, t_W0, t_b0,
           att_W1, att_b1, t_W1, t_b1,
           r1_W, r1_b, r2_W, r2_b,
           e1_W, e1_b, e2_W, e2_b):
    src = jnp.pad(edge_index[0].astype(jnp.int32), (0, E_PAD - E))
    dst = jnp.pad(edge_index[1].astype(jnp.int32), (0, E_PAD - E))
    zer = jnp.zeros((N_PAD, D), jnp.float32)

    w2_0, b2_0 = _att_proj_mats(att_W0, att_b0)
    w2_1, b2_1 = _att_proj_mats(att_W1, att_b1)

    h0, a0 = _encode(node_feats, enc_W, enc_b.reshape(1, D), w2_0, b2_0)

    w0 = _edge_scores(a0[:, 0], a0[:, 1], src, dst)
    att0_2d = _softmax(w0[:E].reshape(E // D, D))
    att0 = jnp.pad(att0_2d.reshape(-1), (0, E_PAD - E))
    p = _spmm(h0, src, dst, att0, zer)
    h1, a1 = _layer(p[0], p[1], t_W0, t_b0.reshape(1, D), h0, w2_1, b2_1)

    w1 = _edge_scores(a1[:, 0], a1[:, 1], src, dst)
    att1_2d, asum_2d = _softmax2(w1[:E].reshape(E // D, D), att0_2d)
    att1 = jnp.pad(att1_2d.reshape(-1), (0, E_PAD - E))
    p = _spmm(h1, src, dst, att1, zer)
    h2, u, v, pool = _final(p[0], p[1], t_W1, t_b1.reshape(1, D), h1,
                            e1_W[:D], e1_b.reshape(1, D), e1_W[D:])

    gs = _readout(pool, r1_W, r1_b.reshape(1, D), r2_W.reshape(1, D),
                  r2_b.reshape(1, 1))

    asum = jnp.pad(asum_2d.reshape(-1), (0, E_PAD - E))
    imp = _edge_imp(u, v, src, dst, asum, e2_W.reshape(D),
                    jnp.broadcast_to(e2_b, (L,)))

    return (gs.reshape(1), h2, imp[:E])
